# Initial kernel scaffold; baseline (speedup 1.0000x reference)
import jax
import jax.numpy as jnp
from jax.experimental import pallas as pl


def kernel(token_ids, weight):
    raise NotImplementedError("write your pallas kernel here")


# speedup vs baseline: 1.8469x; 1.8469x over previous
import functools

import jax
import jax.numpy as jnp
from jax import lax
from jax.experimental import pallas as pl
from jax.experimental.pallas import tpu as pltpu
from jax.experimental.pallas import tpu_sc as plsc


def _build_lookup(B, V, D, CH):
    info = plsc.get_sparse_core_info()
    NC, NS = info.num_cores, info.num_subcores
    NW = NC * NS
    b_per_w = B // NW
    n_chunks = b_per_w // CH
    mesh = plsc.VectorSubcoreMesh(core_axis_name="c", subcore_axis_name="s")

    @functools.partial(
        pl.kernel,
        mesh=mesh,
        out_type=jax.ShapeDtypeStruct((B, D), jnp.float32),
        compiler_params=pltpu.CompilerParams(use_tc_tiling_on_sc=False),
        scratch_types=[
            pltpu.VMEM((CH,), jnp.int32),
            pltpu.VMEM((CH, D), jnp.float32),
            pltpu.SemaphoreType.DMA,
        ],
    )
    def lookup(idx_hbm, table_hbm, out_hbm, idx_v, rows_v, sem):
        wid = lax.axis_index("s") * NC + lax.axis_index("c")
        base = pl.multiple_of(wid * b_per_w, 8)

        def body(i, carry):
            off = pl.multiple_of(base + i * CH, 8)
            pltpu.sync_copy(idx_hbm.at[pl.ds(off, CH)], idx_v)
            pltpu.async_copy(table_hbm.at[idx_v], rows_v, sem).wait()
            pltpu.sync_copy(rows_v, out_hbm.at[pl.ds(off, CH)])
            return carry

        lax.fori_loop(0, n_chunks, body, 0)

    return lookup


def kernel(token_ids, weight):
    V, D = weight.shape
    B = token_ids.shape[0] * token_ids.shape[1]
    flat = token_ids.reshape(B).astype(jnp.int32)
    out = _build_lookup(B, V, D, 1024)(flat, weight)
    return out.reshape(token_ids.shape + (D,))


# speedup vs baseline: 1.8602x; 1.0072x over previous
import functools

import jax
import jax.numpy as jnp
from jax import lax
from jax.experimental import pallas as pl
from jax.experimental.pallas import tpu as pltpu
from jax.experimental.pallas import tpu_sc as plsc


def _build_lookup(B, V, D, CH):
    info = plsc.get_sparse_core_info()
    NC, NS = info.num_cores, info.num_subcores
    NW = NC * NS
    b_per_w = B // NW
    n_chunks = b_per_w // CH
    n_pairs = n_chunks // 2
    mesh = plsc.VectorSubcoreMesh(core_axis_name="c", subcore_axis_name="s")

    @functools.partial(
        pl.kernel,
        mesh=mesh,
        out_type=jax.ShapeDtypeStruct((B, D), jnp.float32),
        compiler_params=pltpu.CompilerParams(use_tc_tiling_on_sc=False),
        scratch_types=[
            pltpu.VMEM((b_per_w,), jnp.int32),
            pltpu.VMEM((CH, D), jnp.float32),
            pltpu.VMEM((CH, D), jnp.float32),
            pltpu.SemaphoreType.DMA,
            pltpu.SemaphoreType.DMA,
            pltpu.SemaphoreType.DMA,
            pltpu.SemaphoreType.DMA,
        ],
    )
    def lookup(idx_hbm, table_hbm, out_hbm, idx_v, rows0, rows1,
               sem_g0, sem_g1, sem_s0, sem_s1):
        wid = lax.axis_index("s") * NC + lax.axis_index("c")
        base = pl.multiple_of(wid * b_per_w, 8)
        pltpu.sync_copy(idx_hbm.at[pl.ds(base, b_per_w)], idx_v)

        def fire_gather(local_off, rows, sem):
            idx_slice = idx_v.at[pl.ds(pl.multiple_of(local_off, 8), CH)]
            pltpu.async_copy(table_hbm.at[idx_slice], rows, sem)

        def wait_gather(local_off, rows, sem):
            idx_slice = idx_v.at[pl.ds(pl.multiple_of(local_off, 8), CH)]
            pltpu.make_async_copy(table_hbm.at[idx_slice], rows, sem).wait()

        def fire_store(local_off, rows, sem):
            dst = out_hbm.at[pl.ds(pl.multiple_of(base + local_off, 8), CH)]
            pltpu.async_copy(rows, dst, sem)

        def wait_store(local_off, rows, sem):
            dst = out_hbm.at[pl.ds(pl.multiple_of(base + local_off, 8), CH)]
            pltpu.make_async_copy(rows, dst, sem).wait()

        fire_gather(0, rows0, sem_g0)
        fire_gather(CH, rows1, sem_g1)

        def body(i, carry):
            g = i * 2 * CH
            wait_gather(g, rows0, sem_g0)
            fire_store(g, rows0, sem_s0)
            wait_gather(g + CH, rows1, sem_g1)
            fire_store(g + CH, rows1, sem_s1)
            wait_store(g, rows0, sem_s0)
            fire_gather(g + 2 * CH, rows0, sem_g0)
            wait_store(g + CH, rows1, sem_s1)
            fire_gather(g + 3 * CH, rows1, sem_g1)
            return carry

        lax.fori_loop(0, n_pairs - 1, body, 0)

        g = (n_pairs - 1) * 2 * CH
        wait_gather(g, rows0, sem_g0)
        fire_store(g, rows0, sem_s0)
        wait_gather(g + CH, rows1, sem_g1)
        fire_store(g + CH, rows1, sem_s1)
        wait_store(g, rows0, sem_s0)
        wait_store(g + CH, rows1, sem_s1)

    return lookup


def kernel(token_ids, weight):
    V, D = weight.shape
    B = token_ids.shape[0] * token_ids.shape[1]
    flat = token_ids.reshape(B).astype(jnp.int32)
    out = _build_lookup(B, V, D, 800)(flat, weight)
    return out.reshape(token_ids.shape + (D,))
